# baseline (speedup 1.0000x reference)
import functools

import jax
import jax.numpy as jnp
from jax import lax
from jax.experimental import pallas as pl
from jax.experimental.pallas import tpu as pltpu
from jax.experimental.pallas import tpu_sc as plsc

_NW = 32
_RCH = 64
_S_SC = 768
_CHS = 128
_E2C = 64


def _sc_body(td_ref, mask_ref, ll_ref, out_ref,
             buf0, buf1, tdb, mkb, wb, accv, sems, *, b_dim, nk, inv_n):
    sh = _S_SC // (_NW // b_dim)
    nchunk = sh // _RCH

    wid = lax.axis_index("s") * 2 + lax.axis_index("c")
    b = wid // (_NW // b_dim)
    s0 = (wid % (_NW // b_dim)) * sh

    bufs = (buf0, buf1)

    def cpy(c, slot):
        return pltpu.make_async_copy(
            ll_ref.at[b, pl.ds(s0 + c * _RCH, _RCH)], bufs[slot], sems.at[slot])

    pltpu.sync_copy(td_ref.at[b, pl.ds(s0, sh)], tdb)
    pltpu.sync_copy(mask_ref.at[b, pl.ds(s0, sh)], mkb)
    cpy(0, 0).start()
    for m in range(sh // 16):
        sl = pl.ds(m * 16, 16)
        wb[sl] = tdb[sl] * mkb[sl] * inv_n

    nacc = 4
    total = jnp.zeros((16,), jnp.float32)
    for c in range(nchunk):
        slot = c & 1
        cpy(c, slot).wait()
        if c + 1 < nchunk:
            cpy(c + 1, 1 - slot).start()
        buf = bufs[slot]

        def rbody(r, tot, c=c, buf=buf):
            wsplat = plsc.load_gather(
                wb, [jnp.full((16,), c * _RCH + r, jnp.int32)])
            accs = [jnp.zeros((16,), jnp.float32) for _ in range(nacc)]
            for j in range(nk // 16):
                accs[j % nacc] = accs[j % nacc] + buf[r, pl.ds(j * 16, 16)]
            rowsum = (accs[0] + accs[1]) + (accs[2] + accs[3])
            return tot + rowsum * wsplat

        total = lax.fori_loop(0, _RCH, rbody, total)

    accv[...] = total
    pltpu.sync_copy(accv, out_ref.at[wid])


def _tc_ll_body(td_ref, mask_ref, ll_ref, ne_ref, acc_ne, *, inv_n):
    i = pl.program_id(0)
    nsteps = pl.num_programs(0)

    @pl.when(i == 0)
    def _init():
        acc_ne[...] = jnp.zeros_like(acc_ne)

    w = td_ref[...] * mask_ref[...] * inv_n
    acc_ne[...] += lax.dot_general(
        w, ll_ref[...],
        dimension_numbers=(((1,), (1,)), ((0,), (0,))),
        preferred_element_type=jnp.float32,
    )

    @pl.when(i == nsteps - 1)
    def _fini():
        ne_ref[0, 0] = jnp.sum(acc_ne[...])


def _tc_ev_body(mask_ref, lae_ref, ltm_ref, e2_ref, ev_ref, cnt_ref, acc_ev, *, k):
    i = pl.program_id(0)
    nsteps = pl.num_programs(0)

    @pl.when(i == 0)
    def _init():
        acc_ev[...] = jnp.zeros_like(acc_ev)
        cnt_ref[0, 0] = jnp.float32(0.0)

    mch = mask_ref[...]
    x = lae_ref[...] * ltm_ref[...]
    half = _CHS * k // 2
    for h in range(2):
        ev_l = jnp.dot(x[:, h * half:(h + 1) * half], e2_ref[...],
                       preferred_element_type=jnp.float32)
        mh = mch[:, h * _E2C:(h + 1) * _E2C]
        acc_ev[:, h * _E2C:(h + 1) * _E2C] += jnp.log(jnp.where(mh > 0, ev_l, 1.0))
    cnt_ref[0, 0] += jnp.sum(mch)

    @pl.when(i == nsteps - 1)
    def _fini():
        ev_ref[0, 0] = jnp.sum(acc_ev[...])


def kernel(time_delta_seq, lambda_at_event, lambdas_loss_samples, seq_mask, lambda_type_mask):
    B, S, N, K = lambdas_loss_samples.shape
    NK = N * K
    ll = lambdas_loss_samples.reshape(B, S, NK)
    laef = lambda_at_event.reshape(B, S * K)
    ltmf = lambda_type_mask.reshape(B, S * K)
    maskf = seq_mask.astype(jnp.float32)

    mesh = plsc.VectorSubcoreMesh(core_axis_name="c", subcore_axis_name="s")
    sc_fn = functools.partial(_sc_body, b_dim=B, nk=NK, inv_n=1.0 / N)
    sh = _S_SC // (_NW // B)
    ne_parts = pl.kernel(
        sc_fn,
        out_type=jax.ShapeDtypeStruct((_NW, 16), jnp.float32),
        mesh=mesh,
        compiler_params=pltpu.CompilerParams(needs_layout_passes=False),
        scratch_types=[
            pltpu.VMEM((_RCH, NK), jnp.float32),
            pltpu.VMEM((_RCH, NK), jnp.float32),
            pltpu.VMEM((sh,), jnp.float32),
            pltpu.VMEM((sh,), jnp.float32),
            pltpu.VMEM((sh,), jnp.float32),
            pltpu.VMEM((16,), jnp.float32),
            pltpu.SemaphoreType.DMA((2,)),
        ],
    )(time_delta_seq, maskf, ll)

    off = _S_SC // _CHS
    ne_tc, = pl.pallas_call(
        functools.partial(_tc_ll_body, inv_n=1.0 / N),
        grid=((S - _S_SC) // _CHS,),
        in_specs=[
            pl.BlockSpec((B, _CHS), lambda i: (0, off + i)),
            pl.BlockSpec((B, _CHS), lambda i: (0, off + i)),
            pl.BlockSpec((B, _CHS, NK), lambda i: (0, off + i, 0)),
        ],
        out_specs=[pl.BlockSpec(memory_space=pltpu.SMEM)],
        out_shape=[jax.ShapeDtypeStruct((1, 1), jnp.float32)],
        scratch_shapes=[pltpu.VMEM((B, NK), jnp.float32)],
    )(time_delta_seq, maskf, ll)

    e2 = jnp.kron(jnp.eye(_E2C, dtype=jnp.float32), jnp.ones((K, 1), jnp.float32))
    ev, cnt = pl.pallas_call(
        functools.partial(_tc_ev_body, k=K),
        grid=(S // _CHS,),
        in_specs=[
            pl.BlockSpec((B, _CHS), lambda i: (0, i)),
            pl.BlockSpec((B, _CHS * K), lambda i: (0, i)),
            pl.BlockSpec((B, _CHS * K), lambda i: (0, i)),
            pl.BlockSpec((_E2C * K, _E2C), lambda i: (0, 0)),
        ],
        out_specs=[
            pl.BlockSpec(memory_space=pltpu.SMEM),
            pl.BlockSpec(memory_space=pltpu.SMEM),
        ],
        out_shape=[
            jax.ShapeDtypeStruct((1, 1), jnp.float32),
            jax.ShapeDtypeStruct((1, 1), jnp.float32),
        ],
        scratch_shapes=[pltpu.VMEM((B, _CHS), jnp.float32)],
    )(maskf, laef, ltmf, e2)

    return (ev[0, 0], ne_tc[0, 0] + jnp.sum(ne_parts), cnt[0, 0].astype(jnp.int32))


# speedup vs baseline: 1.4073x; 1.4073x over previous
import functools

import jax
import jax.numpy as jnp
from jax import lax
from jax.experimental import pallas as pl
from jax.experimental.pallas import tpu as pltpu


def _body(td_ref, mask_ref, lae_ref, ltm_ref, ll_ref,
          ev_ref, ne_ref, cnt_ref,
          acc_ne, acc_ev, acc_cnt, *, inv_n):
    i = pl.program_id(0)

    @pl.when(i == 0)
    def _init():
        acc_ne[...] = jnp.zeros_like(acc_ne)
        acc_ev[...] = jnp.zeros_like(acc_ev)
        acc_cnt[...] = jnp.zeros_like(acc_cnt)

    maskf = mask_ref[...]
    w = td_ref[...] * maskf * inv_n
    acc_ne[...] += lax.dot_general(
        w, ll_ref[...],
        dimension_numbers=(((1,), (1,)), ((0,), (0,))),
        preferred_element_type=jnp.float32,
    )

    ev_l = jnp.sum(lae_ref[...] * ltm_ref[...], axis=2)
    acc_ev[...] += jnp.log(jnp.where(maskf > 0, ev_l, 1.0))
    acc_cnt[...] += maskf

    @pl.when(i == pl.num_programs(0) - 1)
    def _fini():
        ne_ref[0, 0] = jnp.sum(acc_ne[...])
        ev_ref[0, 0] = jnp.sum(acc_ev[...])
        cnt_ref[0, 0] = jnp.sum(acc_cnt[...]).astype(jnp.int32)


def kernel(time_delta_seq, lambda_at_event, lambdas_loss_samples, seq_mask, lambda_type_mask):
    B, S, N, K = lambdas_loss_samples.shape
    NK = N * K
    ll = lambdas_loss_samples.reshape(B, S, NK)
    maskf = seq_mask.astype(jnp.float32)

    CH = 256
    grid = (S // CH,)

    body = functools.partial(_body, inv_n=1.0 / N)
    ev, ne, cnt = pl.pallas_call(
        body,
        grid=grid,
        in_specs=[
            pl.BlockSpec((B, CH), lambda i: (0, i)),
            pl.BlockSpec((B, CH), lambda i: (0, i)),
            pl.BlockSpec((B, CH, K), lambda i: (0, i, 0)),
            pl.BlockSpec((B, CH, K), lambda i: (0, i, 0)),
            pl.BlockSpec((B, CH, NK), lambda i: (0, i, 0)),
        ],
        out_specs=[
            pl.BlockSpec(memory_space=pltpu.SMEM),
            pl.BlockSpec(memory_space=pltpu.SMEM),
            pl.BlockSpec(memory_space=pltpu.SMEM),
        ],
        out_shape=[
            jax.ShapeDtypeStruct((1, 1), jnp.float32),
            jax.ShapeDtypeStruct((1, 1), jnp.float32),
            jax.ShapeDtypeStruct((1, 1), jnp.int32),
        ],
        scratch_shapes=[
            pltpu.VMEM((B, NK), jnp.float32),
            pltpu.VMEM((B, CH), jnp.float32),
            pltpu.VMEM((B, CH), jnp.float32),
        ],
    )(time_delta_seq, maskf, lambda_at_event, lambda_type_mask, ll)

    return (ev[0, 0], ne[0, 0], cnt[0, 0])
